# baseline (speedup 1.0000x reference)
import functools

import jax
import jax.numpy as jnp
import numpy as np
from jax.experimental import pallas as pl
from jax.experimental.pallas import tpu as pltpu
from jax.experimental.pallas import tpu_sc as plsc

N = 8192
B = 16
K = 20
EPS = 1e-5
_BN = float(1.0 / np.sqrt(1.0 + EPS))
_IMAX = np.int32(np.iinfo(np.int32).max)

_PC = pl.pallas_call


_R_KNN = 256
_C_KNN = 512
_T_KNN = N // _C_KNN

_IDX_MASK = np.int32(N - 1)
_BIGD = np.float32(1e37)
_SENT = np.float32(3e38)


def _make_knn_body(d):
    R, C = _R_KNN, _C_KNN

    def body(xr_ref, br_ref, xt_ref, bc_ref, lo_ref, hi_ref, out_ref):
        i = pl.program_id(0)
        xr = xr_ref[...]
        sqr = jnp.sum(xr * xr, axis=1, keepdims=True)
        br = br_ref[...]
        lanes = jax.lax.broadcasted_iota(jnp.int32, (R, 128), 1)

        def chunk(j, tk):
            xc = xt_ref[j]
            bc = bc_ref[j]
            sqc = jnp.sum(xc * xc, axis=0, keepdims=True)
            dist = sqr + sqc - 2.0 * jnp.dot(
                xr, xc, preferred_element_type=jnp.float32)
            dist = jnp.maximum(dist, np.float32(1e-30))
            dist = jnp.where(br != bc, _BIGD, dist)
            gcol = j * C + jax.lax.broadcasted_iota(jnp.int32, (R, C), 1)
            key = jax.lax.bitcast_convert_type(
                (jax.lax.bitcast_convert_type(dist, jnp.int32) & ~_IDX_MASK)
                | gcol, jnp.float32)
            q = [key[:, c0:c0 + 128] for c0 in range(0, C, 128)] + [tk]
            for a, b_ in ((0, 3), (1, 4), (0, 2), (1, 3), (0, 1), (2, 4),
                          (1, 2), (3, 4), (2, 3)):
                q[a], q[b_] = (jnp.minimum(q[a], q[b_]),
                               jnp.maximum(q[a], q[b_]))
            q0, q1, q2, q3, q4 = q
            ntk = jnp.full((R, 128), _SENT, jnp.float32)
            for t in range(K):
                m = jnp.min(q0, axis=1, keepdims=True)
                ntk = jnp.where(lanes == t, m, ntk)
                eq = q0 == m
                q0 = jnp.where(eq, q1, q0)
                q1 = jnp.where(eq, q2, q1)
                q2 = jnp.where(eq, q3, q2)
                q3 = jnp.where(eq, q4, q3)
                q4 = jnp.where(eq, _SENT, q4)
            return ntk

        tk0 = jnp.full((R, 128), _SENT, jnp.float32)
        tk = jax.lax.fori_loop(lo_ref[i], hi_ref[i], chunk, tk0)
        out_ref[...] = (jax.lax.bitcast_convert_type(tk, jnp.int32)[:, :K]
                        & _IDX_MASK)

    return body


def _knn(x, br2, xt3, bc3, lo, hi, d):
    nrt = N // _R_KNN
    return _PC(
        _make_knn_body(d),
        grid=(nrt,),
        in_specs=[
            pl.BlockSpec((_R_KNN, d), lambda i: (i, 0)),
            pl.BlockSpec((_R_KNN, 1), lambda i: (i, 0)),
            pl.BlockSpec((_T_KNN, d, _C_KNN), lambda i: (0, 0, 0)),
            pl.BlockSpec((_T_KNN, 1, _C_KNN), lambda i: (0, 0, 0)),
            pl.BlockSpec(memory_space=pltpu.SMEM),
            pl.BlockSpec(memory_space=pltpu.SMEM),
        ],
        out_specs=pl.BlockSpec((_R_KNN, K), lambda i: (i, 0)),
        out_shape=jax.ShapeDtypeStruct((N, K), jnp.int32),
    )(x, br2, xt3, bc3, lo, hi)


_R_MLP = 256


def _emlp_body(x_ref, pg_ref, wu_ref, bu_ref, wv_ref, w2_ref, b2_ref,
               w3_ref, b3_ref, w4u_ref, b4_ref, w4v_ref,
               x1_ref, u2_ref, v2_ref):
    R = _R_MLP
    x = x_ref[...]
    u1 = jnp.dot(x, wu_ref[...],
                 preferred_element_type=jnp.float32) + bu_ref[...]
    pg = pg_ref[...].reshape(K * R, 16)
    v1 = jnp.dot(pg, wv_ref[...],
                 preferred_element_type=jnp.float32)
    u1t = jnp.broadcast_to(u1[None], (K, R, 64)).reshape(K * R, 64)
    h = jnp.maximum(u1t + v1, 0.0) * _BN
    h = jnp.maximum(jnp.dot(h, w2_ref[...],
                            preferred_element_type=jnp.float32)
                    + b2_ref[...], 0.0) * _BN
    h = jnp.maximum(jnp.dot(h, w3_ref[...],
                            preferred_element_type=jnp.float32)
                    + b3_ref[...], 0.0) * _BN
    acc = h[:R]
    for k in range(1, K):
        acc = jnp.maximum(acc, h[k * R:(k + 1) * R])
    x1_ref[...] = acc
    u2_ref[...] = jnp.dot(acc, w4u_ref[...],
                          preferred_element_type=jnp.float32) + b4_ref[...]
    v2_ref[...] = jnp.dot(acc, w4v_ref[...],
                          preferred_element_type=jnp.float32)


def _emlp(x16, posg, w1u, b1, w1v, w2, b2, w3, b3, w4u, b4, w4v):
    R = _R_MLP
    return _PC(
        _emlp_body,
        grid=(N // R,),
        in_specs=[
            pl.BlockSpec((R, 16), lambda i: (i, 0)),
            pl.BlockSpec((K, R, 16), lambda i: (0, i, 0)),
            pl.BlockSpec((16, 64), lambda i: (0, 0)),
            pl.BlockSpec((1, 64), lambda i: (0, 0)),
            pl.BlockSpec((16, 64), lambda i: (0, 0)),
            pl.BlockSpec((64, 64), lambda i: (0, 0)),
            pl.BlockSpec((1, 64), lambda i: (0, 0)),
            pl.BlockSpec((64, 64), lambda i: (0, 0)),
            pl.BlockSpec((1, 64), lambda i: (0, 0)),
            pl.BlockSpec((64, 128), lambda i: (0, 0)),
            pl.BlockSpec((1, 128), lambda i: (0, 0)),
            pl.BlockSpec((64, 128), lambda i: (0, 0)),
        ],
        out_specs=[
            pl.BlockSpec((R, 64), lambda i: (i, 0)),
            pl.BlockSpec((R, 128), lambda i: (i, 0)),
            pl.BlockSpec((R, 128), lambda i: (i, 0)),
        ],
        out_shape=[
            jax.ShapeDtypeStruct((N, 64), jnp.float32),
            jax.ShapeDtypeStruct((N, 128), jnp.float32),
            jax.ShapeDtypeStruct((N, 128), jnp.float32),
        ],
    )(x16, posg, w1u, b1, w1v, w2, b2, w3, b3, w4u, b4, w4v)


_NC = 2
_NS = 16
_NW = _NC * _NS


def _sc_gather(table, idx_flat):
    E = idx_flat.shape[0]
    D = table.shape[1]
    per_w = E // _NW
    CH = 128
    n_ch = per_w // CH
    mesh = plsc.VectorSubcoreMesh(core_axis_name="c", subcore_axis_name="s")

    @functools.partial(
        pl.kernel,
        out_type=jax.ShapeDtypeStruct((E, D), jnp.float32),
        mesh=mesh,
        scratch_types=[
            pltpu.VMEM((per_w,), jnp.int32),
            pltpu.VMEM((2, CH, D), jnp.float32),
            pltpu.SemaphoreType.DMA,
            pltpu.SemaphoreType.DMA,
        ],
        compiler_params=pltpu.CompilerParams(use_tc_tiling_on_sc=False),
    )
    def k(table_hbm, idx_hbm, out_hbm, idx_v, rows_v, sem0, sem1):
        wid = jax.lax.axis_index("s") * _NC + jax.lax.axis_index("c")
        base = wid * per_w
        sems = (sem0, sem1)
        pltpu.sync_copy(idx_hbm.at[pl.ds(base, per_w)], idx_v)

        def fire(c, b):
            pltpu.make_async_copy(
                table_hbm.at[idx_v.at[pl.ds(c * CH, CH)]], rows_v.at[b],
                sems[b]).start()

        def drain_store(c, b):
            pltpu.make_async_copy(
                table_hbm.at[idx_v.at[pl.ds(c * CH, CH)]], rows_v.at[b],
                sems[b]).wait()
            pltpu.sync_copy(rows_v.at[b], out_hbm.at[pl.ds(base + c * CH, CH)])

        fire(0, 0)

        def step(i2, carry):
            c0 = 2 * i2
            fire(c0 + 1, 1)
            drain_store(c0, 0)

            @pl.when(c0 + 2 < n_ch)
            def _():
                fire(c0 + 2, 0)

            drain_store(c0 + 1, 1)
            return carry

        jax.lax.fori_loop(0, n_ch // 2, step, 0)

    return k(table, idx_flat)


def _sc_gather_max(v2, u2, idx_flat):
    D = 128
    P = 4
    per_w = N // _NW
    n_ch = per_w // P
    mesh = plsc.VectorSubcoreMesh(core_axis_name="c", subcore_axis_name="s")

    @functools.partial(
        pl.kernel,
        out_type=jax.ShapeDtypeStruct((N, D), jnp.float32),
        mesh=mesh,
        scratch_types=[
            pltpu.VMEM((per_w * K,), jnp.int32),
            pltpu.VMEM((4, P * K, D), jnp.float32),
            pltpu.VMEM((per_w, D), jnp.float32),
            pltpu.VMEM((per_w, D), jnp.float32),
            pltpu.SemaphoreType.DMA,
            pltpu.SemaphoreType.DMA,
            pltpu.SemaphoreType.DMA,
            pltpu.SemaphoreType.DMA,
        ],
    )
    def k(v2_hbm, u2_hbm, idx_hbm, out_hbm, idx_v, rows_v, u_v, o_v,
          gs0, gs1, gs2, gs3):
        wid = jax.lax.axis_index("s") * _NC + jax.lax.axis_index("c")
        pbase = wid * per_w
        gsems = (gs0, gs1, gs2, gs3)
        pltpu.sync_copy(idx_hbm.at[pl.ds(pbase * K, per_w * K)], idx_v)
        pltpu.sync_copy(u2_hbm.at[pl.ds(pbase, per_w)], u_v)

        def fire(ci, b):
            pltpu.make_async_copy(
                v2_hbm.at[idx_v.at[pl.ds(ci * P * K, P * K)]],
                rows_v.at[b], gsems[b]).start()

        def compute(ci, b):
            pltpu.make_async_copy(
                v2_hbm.at[idx_v.at[pl.ds(ci * P * K, P * K)]],
                rows_v.at[b], gsems[b]).wait()
            for p in range(P):
                for l in range(D // 16):
                    sl = pl.ds(l * 16, 16)
                    acc = rows_v[b, p * K, sl]
                    for kk in range(1, K):
                        acc = jnp.maximum(acc, rows_v[b, p * K + kk, sl])
                    o_v[ci * P + p, sl] = jnp.maximum(
                        acc + u_v[ci * P + p, sl], 0.0) * _BN

        fire(0, 0)
        fire(1, 1)
        fire(2, 2)

        def step(i4, carry):
            c0 = 4 * i4
            for b in range(4):
                c = c0 + b

                @pl.when(c + 3 < n_ch)
                def _():
                    fire(c + 3, (b + 3) % 4)

                compute(c, b)
            return carry

        jax.lax.fori_loop(0, n_ch // 4, step, 0)
        pltpu.sync_copy(o_v, out_hbm.at[pl.ds(pbase, per_w)])

    return k(v2, u2, idx_flat)


_R_POOL = 256


def _pool_body(x1_ref, x2_ref, w5a_ref, w5b_ref, b5_ref, br_ref,
               w6_ref, b6_ref, w7_ref, b7_ref, w8_ref, b8_ref,
               pool_ref, out_ref):
    i = pl.program_id(0)
    h = (jnp.dot(x1_ref[...], w5a_ref[...], preferred_element_type=jnp.float32)
         + jnp.dot(x2_ref[...], w5b_ref[...], preferred_element_type=jnp.float32)
         + b5_ref[...])
    h = jnp.maximum(h, 0.0) * _BN
    br = br_ref[...]

    @pl.when(i == 0)
    def _():
        pool_ref[...] = jnp.full((B, 1024), -jnp.inf, jnp.float32)

    rows = [jnp.max(jnp.where(br == b, h, -jnp.inf), axis=0, keepdims=True)
            for b in range(B)]
    pool_ref[...] = jnp.maximum(pool_ref[...], jnp.concatenate(rows, axis=0))

    @pl.when(i == N // _R_POOL - 1)
    def _():
        hh = jnp.maximum(jnp.dot(pool_ref[...], w6_ref[...],
                                 preferred_element_type=jnp.float32)
                         + b6_ref[...], 0.0) * _BN
        hh = jnp.maximum(jnp.dot(hh, w7_ref[...],
                                 preferred_element_type=jnp.float32)
                         + b7_ref[...], 0.0) * _BN
        logits = jnp.dot(hh, w8_ref[...],
                         preferred_element_type=jnp.float32) + b8_ref[...]
        m = jnp.max(logits, axis=-1, keepdims=True)
        e = jnp.exp(logits - m)
        lse = jnp.log(jnp.sum(e, axis=-1, keepdims=True)) + m
        out_ref[...] = logits - lse


def _pool_head(x1, x2, w5a, w5b, b5, br2, W6, b6, W7, b7, W8, b8):
    R = _R_POOL
    nc = W8.shape[1]
    return _PC(
        _pool_body,
        grid=(N // R,),
        in_specs=[
            pl.BlockSpec((R, 64), lambda i: (i, 0)),
            pl.BlockSpec((R, 128), lambda i: (i, 0)),
            pl.BlockSpec((64, 1024), lambda i: (0, 0)),
            pl.BlockSpec((128, 1024), lambda i: (0, 0)),
            pl.BlockSpec((1, 1024), lambda i: (0, 0)),
            pl.BlockSpec((R, 1), lambda i: (i, 0)),
            pl.BlockSpec((1024, 512), lambda i: (0, 0)),
            pl.BlockSpec((1, 512), lambda i: (0, 0)),
            pl.BlockSpec((512, 256), lambda i: (0, 0)),
            pl.BlockSpec((1, 256), lambda i: (0, 0)),
            pl.BlockSpec((256, nc), lambda i: (0, 0)),
            pl.BlockSpec((1, nc), lambda i: (0, 0)),
        ],
        out_specs=[
            pl.BlockSpec((B, 1024), lambda i: (0, 0)),
            pl.BlockSpec((B, nc), lambda i: (0, 0)),
        ],
        out_shape=[
            jax.ShapeDtypeStruct((B, 1024), jnp.float32),
            jax.ShapeDtypeStruct((B, nc), jnp.float32),
        ],
    )(x1, x2, w5a, w5b, b5, br2, W6, b6.reshape(1, -1), W7,
      b7.reshape(1, -1), W8, b8.reshape(1, -1))


def kernel(pos, batch, W1, b1, W2, b2, W3, b3, W4, b4, W5, b5, W6, b6, W7, b7,
           W8, b8):
    batch = batch.astype(jnp.int32)
    C, T = _C_KNN, _T_KNN
    R = _R_KNN

    pos_pad = jnp.pad(pos, ((0, 0), (0, 13)))
    w1u = jnp.pad(W1[:3] - W1[3:], ((0, 13), (0, 0)))
    w1v = jnp.pad(W1[3:], ((0, 13), (0, 0)))
    w4u, w4v = W4[:64] - W4[64:], W4[64:]

    br2 = batch[:, None]
    bc3 = batch.reshape(T, 1, C)
    offsets = jnp.searchsorted(
        batch, jnp.arange(B + 1, dtype=jnp.int32), side="left"
    ).astype(jnp.int32)
    bt = batch.reshape(N // R, R)
    starts = offsets[bt[:, 0]]
    ends = offsets[bt[:, -1] + 1]
    lo = (starts // C).astype(jnp.int32)
    hi = ((ends + C - 1) // C).astype(jnp.int32)

    xt3_1 = pos_pad.T.reshape(16, T, C).transpose(1, 0, 2)
    idx1 = _knn(pos_pad, br2, xt3_1, bc3, lo, hi, 16)

    idx1t = idx1.T.reshape(-1)
    posg = _sc_gather(pos_pad, idx1t).reshape(K, N, 16)
    x1, u2, v2 = _emlp(pos_pad, posg, w1u, b1.reshape(1, -1), w1v,
                       W2, b2.reshape(1, -1), W3, b3.reshape(1, -1),
                       w4u, b4.reshape(1, -1), w4v)

    xt3_2 = x1.T.reshape(64, T, C).transpose(1, 0, 2)
    idx2 = _knn(x1, br2, xt3_2, bc3, lo, hi, 64)
    x2 = _sc_gather_max(v2, u2, idx2.reshape(-1))

    _, out = _pool_head(x1, x2, W5[:64], W5[64:], b5.reshape(1, -1), br2,
                        W6, b6, W7, b7, W8, b8)
    return out


# speedup vs baseline: 1.0142x; 1.0142x over previous
import functools

import jax
import jax.numpy as jnp
import numpy as np
from jax.experimental import pallas as pl
from jax.experimental.pallas import tpu as pltpu
from jax.experimental.pallas import tpu_sc as plsc

N = 8192
B = 16
K = 20
EPS = 1e-5
_BN = float(1.0 / np.sqrt(1.0 + EPS))
_IMAX = np.int32(np.iinfo(np.int32).max)

_PC = pl.pallas_call


_R_KNN = 256
_C_KNN = 512
_T_KNN = N // _C_KNN

_IDX_MASK = np.int32(N - 1)
_BIGD = np.float32(1e37)
_SENT = np.float32(3e38)


def _make_knn_body(d):
    R, C = _R_KNN, _C_KNN

    def body(xr_ref, br_ref, xt_ref, bc_ref, lo_ref, hi_ref, out_ref):
        i = pl.program_id(0)
        xr = xr_ref[...]
        sqr = jnp.sum(xr * xr, axis=1, keepdims=True)
        br = br_ref[...]
        lanes = jax.lax.broadcasted_iota(jnp.int32, (R, 128), 1)

        def chunk(j, tk):
            xc = xt_ref[j]
            bc = bc_ref[j]
            sqc = jnp.sum(xc * xc, axis=0, keepdims=True)
            dist = sqr + sqc - 2.0 * jnp.dot(
                xr, xc, preferred_element_type=jnp.float32)
            dist = jnp.maximum(dist, np.float32(1e-30))
            dist = jnp.where(br != bc, _BIGD, dist)
            gcol = j * C + jax.lax.broadcasted_iota(jnp.int32, (R, C), 1)
            key = jax.lax.bitcast_convert_type(
                (jax.lax.bitcast_convert_type(dist, jnp.int32) & ~_IDX_MASK)
                | gcol, jnp.float32)
            q = [key[:, c0:c0 + 128] for c0 in range(0, C, 128)] + [tk]
            for a, b_ in ((0, 3), (1, 4), (0, 2), (1, 3), (0, 1), (2, 4),
                          (1, 2), (3, 4), (2, 3)):
                q[a], q[b_] = (jnp.minimum(q[a], q[b_]),
                               jnp.maximum(q[a], q[b_]))
            q0, q1, q2, q3, q4 = q
            ntk = jnp.full((R, 128), _SENT, jnp.float32)
            for t in range(K):
                m = jnp.min(q0, axis=1, keepdims=True)
                ntk = jnp.where(lanes == t, m, ntk)
                eq = q0 == m
                q0 = jnp.where(eq, q1, q0)
                q1 = jnp.where(eq, q2, q1)
                q2 = jnp.where(eq, q3, q2)
                q3 = jnp.where(eq, q4, q3)
                q4 = jnp.where(eq, _SENT, q4)
            return ntk

        tk0 = jnp.full((R, 128), _SENT, jnp.float32)
        tk = jax.lax.fori_loop(lo_ref[i], hi_ref[i], chunk, tk0)
        out_ref[...] = (jax.lax.bitcast_convert_type(tk, jnp.int32)[:, :K]
                        & _IDX_MASK)

    return body


def _knn(x, br2, xt3, bc3, lo, hi, d):
    nrt = N // _R_KNN
    return _PC(
        _make_knn_body(d),
        grid=(nrt,),
        in_specs=[
            pl.BlockSpec((_R_KNN, d), lambda i: (i, 0)),
            pl.BlockSpec((_R_KNN, 1), lambda i: (i, 0)),
            pl.BlockSpec((_T_KNN, d, _C_KNN), lambda i: (0, 0, 0)),
            pl.BlockSpec((_T_KNN, 1, _C_KNN), lambda i: (0, 0, 0)),
            pl.BlockSpec(memory_space=pltpu.SMEM),
            pl.BlockSpec(memory_space=pltpu.SMEM),
        ],
        out_specs=pl.BlockSpec((_R_KNN, K), lambda i: (i, 0)),
        out_shape=jax.ShapeDtypeStruct((N, K), jnp.int32),
    )(x, br2, xt3, bc3, lo, hi)


_R_MLP = 256


def _emlp_body(x_ref, pg_ref, wu_ref, bu_ref, wv_ref, w2_ref, b2_ref,
               w3_ref, b3_ref, w4u_ref, b4_ref, w4v_ref,
               x1_ref, u2_ref, v2_ref):
    R = _R_MLP
    x = x_ref[...]
    u1 = jnp.dot(x, wu_ref[...],
                 preferred_element_type=jnp.float32) + bu_ref[...]
    pg = pg_ref[...].reshape(K * R, 16)
    v1 = jnp.dot(pg, wv_ref[...],
                 preferred_element_type=jnp.float32)
    u1t = jnp.broadcast_to(u1[None], (K, R, 64)).reshape(K * R, 64)
    h = jnp.maximum(u1t + v1, 0.0) * _BN
    h = jnp.maximum(jnp.dot(h, w2_ref[...],
                            preferred_element_type=jnp.float32)
                    + b2_ref[...], 0.0) * _BN
    h = jnp.maximum(jnp.dot(h, w3_ref[...],
                            preferred_element_type=jnp.float32)
                    + b3_ref[...], 0.0) * _BN
    acc = h[:R]
    for k in range(1, K):
        acc = jnp.maximum(acc, h[k * R:(k + 1) * R])
    x1_ref[...] = acc
    u2_ref[...] = jnp.dot(acc, w4u_ref[...],
                          preferred_element_type=jnp.float32) + b4_ref[...]
    v2_ref[...] = jnp.dot(acc, w4v_ref[...],
                          preferred_element_type=jnp.float32)


def _emlp(x16, posg, w1u, b1, w1v, w2, b2, w3, b3, w4u, b4, w4v):
    R = _R_MLP
    return _PC(
        _emlp_body,
        grid=(N // R,),
        in_specs=[
            pl.BlockSpec((R, 16), lambda i: (i, 0)),
            pl.BlockSpec((K, R, 16), lambda i: (0, i, 0)),
            pl.BlockSpec((16, 64), lambda i: (0, 0)),
            pl.BlockSpec((1, 64), lambda i: (0, 0)),
            pl.BlockSpec((16, 64), lambda i: (0, 0)),
            pl.BlockSpec((64, 64), lambda i: (0, 0)),
            pl.BlockSpec((1, 64), lambda i: (0, 0)),
            pl.BlockSpec((64, 64), lambda i: (0, 0)),
            pl.BlockSpec((1, 64), lambda i: (0, 0)),
            pl.BlockSpec((64, 128), lambda i: (0, 0)),
            pl.BlockSpec((1, 128), lambda i: (0, 0)),
            pl.BlockSpec((64, 128), lambda i: (0, 0)),
        ],
        out_specs=[
            pl.BlockSpec((R, 64), lambda i: (i, 0)),
            pl.BlockSpec((R, 128), lambda i: (i, 0)),
            pl.BlockSpec((R, 128), lambda i: (i, 0)),
        ],
        out_shape=[
            jax.ShapeDtypeStruct((N, 64), jnp.float32),
            jax.ShapeDtypeStruct((N, 128), jnp.float32),
            jax.ShapeDtypeStruct((N, 128), jnp.float32),
        ],
    )(x16, posg, w1u, b1, w1v, w2, b2, w3, b3, w4u, b4, w4v)


_NC = 2
_NS = 16
_NW = _NC * _NS


def _sc_gather(table, idx_flat):
    E = idx_flat.shape[0]
    D = table.shape[1]
    per_w = E // _NW
    CH = 128
    n_ch = per_w // CH
    mesh = plsc.VectorSubcoreMesh(core_axis_name="c", subcore_axis_name="s")

    @functools.partial(
        pl.kernel,
        out_type=jax.ShapeDtypeStruct((E, D), jnp.float32),
        mesh=mesh,
        scratch_types=[
            pltpu.VMEM((per_w,), jnp.int32),
            pltpu.VMEM((2, CH, D), jnp.float32),
            pltpu.SemaphoreType.DMA,
            pltpu.SemaphoreType.DMA,
        ],
        compiler_params=pltpu.CompilerParams(use_tc_tiling_on_sc=False),
    )
    def k(table_hbm, idx_hbm, out_hbm, idx_v, rows_v, sem0, sem1):
        wid = jax.lax.axis_index("s") * _NC + jax.lax.axis_index("c")
        base = wid * per_w
        sems = (sem0, sem1)
        pltpu.sync_copy(idx_hbm.at[pl.ds(base, per_w)], idx_v)

        def fire(c, b):
            pltpu.make_async_copy(
                table_hbm.at[idx_v.at[pl.ds(c * CH, CH)]], rows_v.at[b],
                sems[b]).start()

        def drain_store(c, b):
            pltpu.make_async_copy(
                table_hbm.at[idx_v.at[pl.ds(c * CH, CH)]], rows_v.at[b],
                sems[b]).wait()
            pltpu.sync_copy(rows_v.at[b], out_hbm.at[pl.ds(base + c * CH, CH)])

        fire(0, 0)

        def step(i2, carry):
            c0 = 2 * i2
            fire(c0 + 1, 1)
            drain_store(c0, 0)

            @pl.when(c0 + 2 < n_ch)
            def _():
                fire(c0 + 2, 0)

            drain_store(c0 + 1, 1)
            return carry

        jax.lax.fori_loop(0, n_ch // 2, step, 0)

    return k(table, idx_flat)


def _sc_gather_max(v2, u2, idx_flat):
    D = 128
    P = 4
    per_w = N // _NW
    n_ch = per_w // P
    mesh = plsc.VectorSubcoreMesh(core_axis_name="c", subcore_axis_name="s")

    @functools.partial(
        pl.kernel,
        out_type=jax.ShapeDtypeStruct((N, D), jnp.float32),
        mesh=mesh,
        scratch_types=[
            pltpu.VMEM((per_w * K,), jnp.int32),
            pltpu.VMEM((2, P * K, D), jnp.float32),
            pltpu.VMEM((per_w, D), jnp.float32),
            pltpu.VMEM((per_w, D), jnp.float32),
            pltpu.SemaphoreType.DMA,
            pltpu.SemaphoreType.DMA,
        ],
    )
    def k(v2_hbm, u2_hbm, idx_hbm, out_hbm, idx_v, rows_v, u_v, o_v,
          gs0, gs1):
        wid = jax.lax.axis_index("s") * _NC + jax.lax.axis_index("c")
        pbase = wid * per_w
        gsems = (gs0, gs1)
        pltpu.sync_copy(idx_hbm.at[pl.ds(pbase * K, per_w * K)], idx_v)
        pltpu.sync_copy(u2_hbm.at[pl.ds(pbase, per_w)], u_v)

        def fire(ci, b):
            pltpu.make_async_copy(
                v2_hbm.at[idx_v.at[pl.ds(ci * P * K, P * K)]],
                rows_v.at[b], gsems[b]).start()

        def compute(ci, b):
            pltpu.make_async_copy(
                v2_hbm.at[idx_v.at[pl.ds(ci * P * K, P * K)]],
                rows_v.at[b], gsems[b]).wait()
            for p in range(P):
                for l in range(D // 16):
                    sl = pl.ds(l * 16, 16)
                    acc = rows_v[b, p * K, sl]
                    for kk in range(1, K):
                        acc = jnp.maximum(acc, rows_v[b, p * K + kk, sl])
                    o_v[ci * P + p, sl] = jnp.maximum(
                        acc + u_v[ci * P + p, sl], 0.0) * _BN

        fire(0, 0)

        def step(i2, carry):
            c0 = 2 * i2
            fire(c0 + 1, 1)
            compute(c0, 0)

            @pl.when(c0 + 2 < n_ch)
            def _():
                fire(c0 + 2, 0)

            compute(c0 + 1, 1)
            return carry

        jax.lax.fori_loop(0, n_ch // 2, step, 0)
        pltpu.sync_copy(o_v, out_hbm.at[pl.ds(pbase, per_w)])

    return k(v2, u2, idx_flat)


_R_POOL = 256


def _pool_body(x1_ref, x2_ref, w5a_ref, w5b_ref, b5_ref, br_ref,
               w6_ref, b6_ref, w7_ref, b7_ref, w8_ref, b8_ref,
               pool_ref, out_ref):
    i = pl.program_id(0)
    h = (jnp.dot(x1_ref[...], w5a_ref[...], preferred_element_type=jnp.float32)
         + jnp.dot(x2_ref[...], w5b_ref[...], preferred_element_type=jnp.float32)
         + b5_ref[...])
    h = jnp.maximum(h, 0.0) * _BN
    br = br_ref[...]

    @pl.when(i == 0)
    def _():
        pool_ref[...] = jnp.full((B, 1024), -jnp.inf, jnp.float32)

    rows = [jnp.max(jnp.where(br == b, h, -jnp.inf), axis=0, keepdims=True)
            for b in range(B)]
    pool_ref[...] = jnp.maximum(pool_ref[...], jnp.concatenate(rows, axis=0))

    @pl.when(i == N // _R_POOL - 1)
    def _():
        hh = jnp.maximum(jnp.dot(pool_ref[...], w6_ref[...],
                                 preferred_element_type=jnp.float32)
                         + b6_ref[...], 0.0) * _BN
        hh = jnp.maximum(jnp.dot(hh, w7_ref[...],
                                 preferred_element_type=jnp.float32)
                         + b7_ref[...], 0.0) * _BN
        logits = jnp.dot(hh, w8_ref[...],
                         preferred_element_type=jnp.float32) + b8_ref[...]
        m = jnp.max(logits, axis=-1, keepdims=True)
        e = jnp.exp(logits - m)
        lse = jnp.log(jnp.sum(e, axis=-1, keepdims=True)) + m
        out_ref[...] = logits - lse


def _pool_head(x1, x2, w5a, w5b, b5, br2, W6, b6, W7, b7, W8, b8):
    R = _R_POOL
    nc = W8.shape[1]
    return _PC(
        _pool_body,
        grid=(N // R,),
        in_specs=[
            pl.BlockSpec((R, 64), lambda i: (i, 0)),
            pl.BlockSpec((R, 128), lambda i: (i, 0)),
            pl.BlockSpec((64, 1024), lambda i: (0, 0)),
            pl.BlockSpec((128, 1024), lambda i: (0, 0)),
            pl.BlockSpec((1, 1024), lambda i: (0, 0)),
            pl.BlockSpec((R, 1), lambda i: (i, 0)),
            pl.BlockSpec((1024, 512), lambda i: (0, 0)),
            pl.BlockSpec((1, 512), lambda i: (0, 0)),
            pl.BlockSpec((512, 256), lambda i: (0, 0)),
            pl.BlockSpec((1, 256), lambda i: (0, 0)),
            pl.BlockSpec((256, nc), lambda i: (0, 0)),
            pl.BlockSpec((1, nc), lambda i: (0, 0)),
        ],
        out_specs=[
            pl.BlockSpec((B, 1024), lambda i: (0, 0)),
            pl.BlockSpec((B, nc), lambda i: (0, 0)),
        ],
        out_shape=[
            jax.ShapeDtypeStruct((B, 1024), jnp.float32),
            jax.ShapeDtypeStruct((B, nc), jnp.float32),
        ],
    )(x1, x2, w5a, w5b, b5, br2, W6, b6.reshape(1, -1), W7,
      b7.reshape(1, -1), W8, b8.reshape(1, -1))


def kernel(pos, batch, W1, b1, W2, b2, W3, b3, W4, b4, W5, b5, W6, b6, W7, b7,
           W8, b8):
    batch = batch.astype(jnp.int32)
    C, T = _C_KNN, _T_KNN
    R = _R_KNN

    pos_pad = jnp.pad(pos, ((0, 0), (0, 13)))
    w1u = jnp.pad(W1[:3] - W1[3:], ((0, 13), (0, 0)))
    w1v = jnp.pad(W1[3:], ((0, 13), (0, 0)))
    w4u, w4v = W4[:64] - W4[64:], W4[64:]

    br2 = batch[:, None]
    bc3 = batch.reshape(T, 1, C)
    offsets = jnp.searchsorted(
        batch, jnp.arange(B + 1, dtype=jnp.int32), side="left"
    ).astype(jnp.int32)
    bt = batch.reshape(N // R, R)
    starts = offsets[bt[:, 0]]
    ends = offsets[bt[:, -1] + 1]
    lo = (starts // C).astype(jnp.int32)
    hi = ((ends + C - 1) // C).astype(jnp.int32)

    xt3_1 = pos_pad.T.reshape(16, T, C).transpose(1, 0, 2)
    idx1 = _knn(pos_pad, br2, xt3_1, bc3, lo, hi, 16)

    idx1t = idx1.T.reshape(-1)
    posg = _sc_gather(pos_pad, idx1t).reshape(K, N, 16)
    x1, u2, v2 = _emlp(pos_pad, posg, w1u, b1.reshape(1, -1), w1v,
                       W2, b2.reshape(1, -1), W3, b3.reshape(1, -1),
                       w4u, b4.reshape(1, -1), w4v)

    xt3_2 = x1.T.reshape(64, T, C).transpose(1, 0, 2)
    idx2 = _knn(x1, br2, xt3_2, bc3, lo, hi, 64)
    x2 = _sc_gather_max(v2, u2, idx2.reshape(-1))

    _, out = _pool_head(x1, x2, W5[:64], W5[64:], b5.reshape(1, -1), br2,
                        W6, b6, W7, b7, W8, b8)
    return out
